# baseline (speedup 1.0000x reference)
import functools

import jax
import jax.numpy as jnp
from jax import lax
from jax.experimental import pallas as pl
from jax.experimental.pallas import tpu as pltpu
from jax.experimental.pallas import tpu_sc as plsc

N = 10000
E = 160000
D_IN = 256
D_HID = 256
D_OUT = 64

NC = 2
NS = 16
B = 125
NB = E // NS // B
NP = 10240
RPT = NP // NS
IB = 16
NIC = NB // IB
NBC = NB // NC

_MESH = dict(core_axis_name="c", subcore_axis_name="s",
             num_cores=NC, num_subcores=NS)

_UNTILED = pltpu.CompilerParams(use_tc_tiling_on_sc=False)


@functools.partial(
    pl.kernel,
    out_type=jax.ShapeDtypeStruct((NC, NP, 16), jnp.float32),
    mesh=plsc.VectorSubcoreMesh(**_MESH),
    compiler_params=_UNTILED,
    scratch_types=[
        pltpu.VMEM((NBC, B), jnp.int32),
        pltpu.VMEM((B, 16), jnp.float32),
        pltpu.VMEM_SHARED((NP, 16), jnp.float32),
    ],
)
def _deg_kernel(ei, ones_h, zz_h, out, rowb, onesb, acc):
    c = lax.axis_index("c")
    s = lax.axis_index("s")
    pltpu.sync_copy(zz_h, acc.at[pl.ds(s * RPT, RPT)])
    pltpu.sync_copy(ones_h, onesb)
    pltpu.sync_copy(ei.at[0, s, pl.ds(c * NBC, NBC)], rowb)
    plsc.subcore_barrier()

    def body(j, _):
        pltpu.sync_copy(onesb, acc.at[rowb.at[j]], add=True)
        return 0

    lax.fori_loop(0, NBC, body, 0)
    plsc.subcore_barrier()
    pltpu.sync_copy(acc.at[pl.ds(s * RPT, RPT)], out.at[c, pl.ds(s * RPT, RPT)])


@functools.partial(
    pl.kernel,
    out_type=jax.ShapeDtypeStruct((NC, NP, 128), jnp.float32),
    mesh=plsc.VectorSubcoreMesh(**_MESH),
    scratch_types=[
        pltpu.VMEM((IB, B), jnp.int32),
        pltpu.VMEM((IB, B), jnp.int32),
        pltpu.VMEM((2, B, 128), jnp.float32),
        pltpu.VMEM_SHARED((NP, 128), jnp.float32),
        pltpu.SemaphoreType.DMA,
        pltpu.SemaphoreType.DMA,
        pltpu.SemaphoreType.DMA,
        pltpu.SemaphoreType.DMA,
    ],
)
def _spmm1(x0, x1, ei, zz_h, out, colb, rowb, gbuf, acc,
           gsem0, gsem1, ssem0, ssem1):
    c = lax.axis_index("c")
    s = lax.axis_index("s")
    pltpu.sync_copy(zz_h, acc.at[pl.ds(s * RPT, RPT)])
    plsc.subcore_barrier()

    gsems = (gsem0, gsem1)
    ssems = (ssem0, ssem1)

    def run(xref):
        def gstart(jj, b):
            pltpu.async_copy(xref.at[colb.at[jj]], gbuf.at[b], gsems[b])

        def gwait(jj, b):
            pltpu.make_async_copy(
                xref.at[colb.at[jj]], gbuf.at[b], gsems[b]).wait()

        def sstart(jj, b):
            pltpu.async_copy(gbuf.at[b], acc.at[rowb.at[jj]], ssems[b],
                             add=True)

        def swait(jj, b):
            pltpu.make_async_copy(gbuf.at[b], acc.at[rowb.at[jj]],
                                  ssems[b]).wait()

        def chunk(ic, _):
            pltpu.sync_copy(ei.at[1, s, pl.ds(ic * IB, IB)], colb)
            pltpu.sync_copy(ei.at[0, s, pl.ds(ic * IB, IB)], rowb)
            gstart(0, 0)

            def pair(i, _):
                jj0 = i * 2
                gwait(jj0, 0)
                sstart(jj0, 0)

                @pl.when(jj0 >= 1)
                def _():
                    swait(jj0 - 1, 1)

                gstart(jj0 + 1, 1)
                gwait(jj0 + 1, 1)
                sstart(jj0 + 1, 1)
                swait(jj0, 0)

                @pl.when(jj0 + 2 < IB)
                def _():
                    gstart(jj0 + 2, 0)
                return 0

            lax.fori_loop(0, IB // 2, pair, 0)
            swait(IB - 1, 1)
            return 0

        lax.fori_loop(0, NIC, chunk, 0)

    @pl.when(c == 0)
    def _():
        run(x0)

    @pl.when(c == 1)
    def _():
        run(x1)

    plsc.subcore_barrier()
    pltpu.sync_copy(acc.at[pl.ds(s * RPT, RPT)], out.at[c, pl.ds(s * RPT, RPT)])


IB2 = 8
NIC2 = NBC // IB2


@functools.partial(
    pl.kernel,
    out_type=jax.ShapeDtypeStruct((NC, NP, D_OUT), jnp.float32),
    mesh=plsc.VectorSubcoreMesh(**_MESH),
    compiler_params=_UNTILED,
    scratch_types=[
        pltpu.VMEM((IB2, B), jnp.int32),
        pltpu.VMEM((IB2, B), jnp.int32),
        pltpu.VMEM((2, B, D_OUT), jnp.float32),
        pltpu.VMEM_SHARED((NP, D_OUT), jnp.float32),
        pltpu.SemaphoreType.DMA,
        pltpu.SemaphoreType.DMA,
    ],
)
def _spmm2(x, ei, zz_h, out, colb, rowb, gbuf, acc, gsem0, gsem1):
    c = lax.axis_index("c")
    s = lax.axis_index("s")
    pltpu.sync_copy(zz_h, acc.at[pl.ds(s * RPT, RPT)])
    plsc.subcore_barrier()

    gsems = (gsem0, gsem1)

    def gstart(jj, b):
        pltpu.async_copy(x.at[colb.at[jj]], gbuf.at[b], gsems[b])

    def gwait(jj, b):
        pltpu.make_async_copy(x.at[colb.at[jj]], gbuf.at[b], gsems[b]).wait()

    def chunk(ic, _):
        base = c * NBC + ic * IB2
        pltpu.sync_copy(ei.at[1, s, pl.ds(base, IB2)], colb)
        pltpu.sync_copy(ei.at[0, s, pl.ds(base, IB2)], rowb)
        gstart(0, 0)
        gstart(1, 1)

        def pair(i, _):
            jj0 = i * 2
            for b in range(2):
                jj = jj0 + b
                gwait(jj, b)
                pltpu.sync_copy(gbuf.at[b], acc.at[rowb.at[jj]], add=True)

                @pl.when(jj + 2 < IB2)
                def _():
                    gstart(jj + 2, b)
            return 0

        lax.fori_loop(0, IB2 // 2, pair, 0)
        return 0

    lax.fori_loop(0, NIC2, chunk, 0)
    plsc.subcore_barrier()
    pltpu.sync_copy(acc.at[pl.ds(s * RPT, RPT)], out.at[c, pl.ds(s * RPT, RPT)])


_RB = 1000
_CB = 1280


def _norm_from(deg2_blk):
    d = deg2_blk[0, :, 0:1] + deg2_blk[1, :, 0:1]
    return lax.rsqrt(jnp.maximum(d, 1.0))


def _mm1_body(h_ref, w_ref, deg_ref, o0_ref, o1_ref):
    nrm = _norm_from(deg_ref[...])
    t = lax.dot_general(h_ref[...], w_ref[...], (((1,), (1,)), ((), ())),
                        preferred_element_type=jnp.float32)
    t = t * nrm
    o0_ref[...] = t[:, :D_HID // 2]
    o1_ref[...] = t[:, D_HID // 2:]


def _mm2_body(a0_ref, a1_ref, w_ref, deg_ref, o_ref):
    nrm = _norm_from(deg_ref[...])
    z = jnp.concatenate([a0_ref[0], a1_ref[0]], axis=1) * nrm
    z = jnp.maximum(z, 0.0)
    t = lax.dot_general(z, w_ref[...], (((1,), (1,)), ((), ())),
                        preferred_element_type=jnp.float32)
    t = t * nrm
    o_ref[...] = t


def _mmc_body(b0_ref, b1_ref, deg_ref, out_ref):
    z = b0_ref[0] + b1_ref[0]
    out_ref[...] = z * _norm_from(deg_ref[...])


def _mm3_body(zi_ref, zj_ref, out_ref):
    out_ref[...] = lax.dot_general(zi_ref[...], zj_ref[...],
                                   (((1,), (1,)), ((), ())),
                                   preferred_element_type=jnp.float32)


def _mm1(h, W0, deg2):
    return pl.pallas_call(
        _mm1_body,
        grid=(N // _RB,),
        in_specs=[
            pl.BlockSpec((_RB, D_IN), lambda i: (i, 0)),
            pl.BlockSpec((D_HID, D_IN), lambda i: (0, 0)),
            pl.BlockSpec((NC, _RB, 16), lambda i: (0, i, 0)),
        ],
        out_specs=[
            pl.BlockSpec((_RB, D_HID // 2), lambda i: (i, 0)),
            pl.BlockSpec((_RB, D_HID // 2), lambda i: (i, 0)),
        ],
        out_shape=[jax.ShapeDtypeStruct((N, D_HID // 2), jnp.float32)] * 2,
    )(h, W0, deg2)


def _mm2(s_all, W1, deg2):
    return pl.pallas_call(
        _mm2_body,
        grid=(N // _RB,),
        in_specs=[
            pl.BlockSpec((1, _RB, 128), lambda i: (0, i, 0)),
            pl.BlockSpec((1, _RB, 128), lambda i: (1, i, 0)),
            pl.BlockSpec((D_OUT, D_HID), lambda i: (0, 0)),
            pl.BlockSpec((NC, _RB, 16), lambda i: (0, i, 0)),
        ],
        out_specs=pl.BlockSpec((_RB, D_OUT), lambda i: (i, 0)),
        out_shape=jax.ShapeDtypeStruct((N, D_OUT), jnp.float32),
    )(s_all, s_all, W1, deg2)


def _mmc(u_all, deg2):
    return pl.pallas_call(
        _mmc_body,
        grid=(N // _RB,),
        in_specs=[
            pl.BlockSpec((1, _RB, D_OUT), lambda i: (0, i, 0)),
            pl.BlockSpec((1, _RB, D_OUT), lambda i: (1, i, 0)),
            pl.BlockSpec((NC, _RB, 16), lambda i: (0, i, 0)),
        ],
        out_specs=pl.BlockSpec((_RB, D_OUT), lambda i: (i, 0)),
        out_shape=jax.ShapeDtypeStruct((N, D_OUT), jnp.float32),
    )(u_all, u_all, deg2)


def _mm3(z):
    ncb = pl.cdiv(N, _CB)
    return pl.pallas_call(
        _mm3_body,
        grid=(N // _RB, ncb),
        in_specs=[
            pl.BlockSpec((_RB, D_OUT), lambda i, j: (i, 0)),
            pl.BlockSpec((_CB, D_OUT), lambda i, j: (j, 0)),
        ],
        out_specs=pl.BlockSpec((_RB, _CB), lambda i, j: (i, j)),
        out_shape=jax.ShapeDtypeStruct((N, N), jnp.float32),
    )(z, z)


def kernel(h, edge_index, W0, W1):
    ei4 = edge_index.reshape(2, NS, NB, B)
    ones16 = jnp.ones((B, 16), jnp.float32)
    zz16 = jnp.zeros((RPT, 16), jnp.float32)
    zz64 = jnp.zeros((RPT, D_OUT), jnp.float32)
    zz128 = jnp.zeros((RPT, 128), jnp.float32)
    deg2 = _deg_kernel(ei4, ones16, zz16)
    x0, x1 = _mm1(h, W0, deg2)
    s_all = _spmm1(x0, x1, ei4, zz128)
    t2 = _mm2(s_all, W1, deg2)
    u_all = _spmm2(t2, ei4, zz64)
    z = _mmc(u_all, deg2)
    return _mm3(z)


# speedup vs baseline: 1.0771x; 1.0771x over previous
import functools

import jax
import jax.numpy as jnp
from jax import lax
from jax.experimental import pallas as pl
from jax.experimental.pallas import tpu as pltpu
from jax.experimental.pallas import tpu_sc as plsc

N = 10000
E = 160000
D_IN = 256
D_HID = 256
D_OUT = 64

NC = 2
NS = 16
B = 125
NB = E // NS // B
NP = 10240
RPT = NP // NS
IB = 16
NIC = NB // IB
NBC = NB // NC

_MESH = dict(core_axis_name="c", subcore_axis_name="s",
             num_cores=NC, num_subcores=NS)

_UNTILED = pltpu.CompilerParams(use_tc_tiling_on_sc=False)


@functools.partial(
    pl.kernel,
    out_type=jax.ShapeDtypeStruct((NC, NP, 16), jnp.float32),
    mesh=plsc.VectorSubcoreMesh(**_MESH),
    compiler_params=_UNTILED,
    scratch_types=[
        pltpu.VMEM((NBC, B), jnp.int32),
        pltpu.VMEM((B, 16), jnp.float32),
        pltpu.VMEM_SHARED((NP, 16), jnp.float32),
    ],
)
def _deg_kernel(ei, ones_h, zz_h, out, rowb, onesb, acc):
    c = lax.axis_index("c")
    s = lax.axis_index("s")
    pltpu.sync_copy(zz_h, acc.at[pl.ds(s * RPT, RPT)])
    pltpu.sync_copy(ones_h, onesb)
    pltpu.sync_copy(ei.at[0, s, pl.ds(c * NBC, NBC)], rowb)
    plsc.subcore_barrier()

    def body(j, _):
        pltpu.sync_copy(onesb, acc.at[rowb.at[j]], add=True)
        return 0

    lax.fori_loop(0, NBC, body, 0)
    plsc.subcore_barrier()
    pltpu.sync_copy(acc.at[pl.ds(s * RPT, RPT)], out.at[c, pl.ds(s * RPT, RPT)])


@functools.partial(
    pl.kernel,
    out_type=jax.ShapeDtypeStruct((NC, NP, 128), jnp.float32),
    mesh=plsc.VectorSubcoreMesh(**_MESH),
    scratch_types=[
        pltpu.VMEM((IB, B), jnp.int32),
        pltpu.VMEM((IB, B), jnp.int32),
        pltpu.VMEM((2, B, 128), jnp.float32),
        pltpu.VMEM_SHARED((NP, 128), jnp.float32),
        pltpu.SemaphoreType.DMA,
        pltpu.SemaphoreType.DMA,
    ],
)
def _spmm1(x0, x1, ei, zz_h, out, colb, rowb, gbuf, acc, gsem0, gsem1):
    c = lax.axis_index("c")
    s = lax.axis_index("s")
    pltpu.sync_copy(zz_h, acc.at[pl.ds(s * RPT, RPT)])
    plsc.subcore_barrier()

    gsems = (gsem0, gsem1)

    def run(xref):
        def gstart(jj, b):
            pltpu.async_copy(xref.at[colb.at[jj]], gbuf.at[b], gsems[b])

        def gwait(jj, b):
            pltpu.make_async_copy(
                xref.at[colb.at[jj]], gbuf.at[b], gsems[b]).wait()

        def chunk(ic, _):
            pltpu.sync_copy(ei.at[1, s, pl.ds(ic * IB, IB)], colb)
            pltpu.sync_copy(ei.at[0, s, pl.ds(ic * IB, IB)], rowb)
            gstart(0, 0)
            gstart(1, 1)

            def pair(i, _):
                jj0 = i * 2
                for b in range(2):
                    jj = jj0 + b
                    gwait(jj, b)
                    pltpu.sync_copy(gbuf.at[b], acc.at[rowb.at[jj]], add=True)

                    @pl.when(jj + 2 < IB)
                    def _():
                        gstart(jj + 2, b)
                return 0

            lax.fori_loop(0, IB // 2, pair, 0)
            return 0

        lax.fori_loop(0, NIC, chunk, 0)

    @pl.when(c == 0)
    def _():
        run(x0)

    @pl.when(c == 1)
    def _():
        run(x1)

    plsc.subcore_barrier()
    pltpu.sync_copy(acc.at[pl.ds(s * RPT, RPT)], out.at[c, pl.ds(s * RPT, RPT)])


IB2 = 8
NIC2 = NBC // IB2


@functools.partial(
    pl.kernel,
    out_type=jax.ShapeDtypeStruct((NC, NP, D_OUT), jnp.float32),
    mesh=plsc.VectorSubcoreMesh(**_MESH),
    compiler_params=_UNTILED,
    scratch_types=[
        pltpu.VMEM((IB2, B), jnp.int32),
        pltpu.VMEM((IB2, B), jnp.int32),
        pltpu.VMEM((2, B, D_OUT), jnp.float32),
        pltpu.VMEM_SHARED((NP, D_OUT), jnp.float32),
        pltpu.SemaphoreType.DMA,
        pltpu.SemaphoreType.DMA,
    ],
)
def _spmm2(x, ei, zz_h, out, colb, rowb, gbuf, acc, gsem0, gsem1):
    c = lax.axis_index("c")
    s = lax.axis_index("s")
    pltpu.sync_copy(zz_h, acc.at[pl.ds(s * RPT, RPT)])
    plsc.subcore_barrier()

    gsems = (gsem0, gsem1)

    def gstart(jj, b):
        pltpu.async_copy(x.at[colb.at[jj]], gbuf.at[b], gsems[b])

    def gwait(jj, b):
        pltpu.make_async_copy(x.at[colb.at[jj]], gbuf.at[b], gsems[b]).wait()

    def chunk(ic, _):
        base = c * NBC + ic * IB2
        pltpu.sync_copy(ei.at[1, s, pl.ds(base, IB2)], colb)
        pltpu.sync_copy(ei.at[0, s, pl.ds(base, IB2)], rowb)
        gstart(0, 0)
        gstart(1, 1)

        def pair(i, _):
            jj0 = i * 2
            for b in range(2):
                jj = jj0 + b
                gwait(jj, b)
                pltpu.sync_copy(gbuf.at[b], acc.at[rowb.at[jj]], add=True)

                @pl.when(jj + 2 < IB2)
                def _():
                    gstart(jj + 2, b)
            return 0

        lax.fori_loop(0, IB2 // 2, pair, 0)
        return 0

    lax.fori_loop(0, NIC2, chunk, 0)
    plsc.subcore_barrier()
    pltpu.sync_copy(acc.at[pl.ds(s * RPT, RPT)], out.at[c, pl.ds(s * RPT, RPT)])


_RB = 1000
_CB = 2560


def _norm_from(deg2_blk):
    d = deg2_blk[0, :, 0:1] + deg2_blk[1, :, 0:1]
    return lax.rsqrt(jnp.maximum(d, 1.0))


def _mm1_body(h_ref, w_ref, deg_ref, o0_ref, o1_ref):
    nrm = _norm_from(deg_ref[...])
    t = lax.dot_general(h_ref[...], w_ref[...], (((1,), (1,)), ((), ())),
                        preferred_element_type=jnp.float32)
    t = t * nrm
    o0_ref[...] = t[:, :D_HID // 2]
    o1_ref[...] = t[:, D_HID // 2:]


def _mm2_body(a0_ref, a1_ref, w_ref, deg_ref, o_ref):
    nrm = _norm_from(deg_ref[...])
    z = jnp.concatenate([a0_ref[0], a1_ref[0]], axis=1) * nrm
    z = jnp.maximum(z, 0.0)
    t = lax.dot_general(z, w_ref[...], (((1,), (1,)), ((), ())),
                        preferred_element_type=jnp.float32)
    t = t * nrm
    o_ref[...] = t


def _mmc_body(b0_ref, b1_ref, deg_ref, out_ref):
    z = b0_ref[0] + b1_ref[0]
    out_ref[...] = z * _norm_from(deg_ref[...])


def _mm3_body(zi_ref, zj_ref, out_ref):
    out_ref[...] = lax.dot_general(zi_ref[...], zj_ref[...],
                                   (((1,), (1,)), ((), ())),
                                   preferred_element_type=jnp.float32)


def _mm1(h, W0, deg2):
    return pl.pallas_call(
        _mm1_body,
        grid=(N // _RB,),
        in_specs=[
            pl.BlockSpec((_RB, D_IN), lambda i: (i, 0)),
            pl.BlockSpec((D_HID, D_IN), lambda i: (0, 0)),
            pl.BlockSpec((NC, _RB, 16), lambda i: (0, i, 0)),
        ],
        out_specs=[
            pl.BlockSpec((_RB, D_HID // 2), lambda i: (i, 0)),
            pl.BlockSpec((_RB, D_HID // 2), lambda i: (i, 0)),
        ],
        out_shape=[jax.ShapeDtypeStruct((N, D_HID // 2), jnp.float32)] * 2,
    )(h, W0, deg2)


def _mm2(s_all, W1, deg2):
    return pl.pallas_call(
        _mm2_body,
        grid=(N // _RB,),
        in_specs=[
            pl.BlockSpec((1, _RB, 128), lambda i: (0, i, 0)),
            pl.BlockSpec((1, _RB, 128), lambda i: (1, i, 0)),
            pl.BlockSpec((D_OUT, D_HID), lambda i: (0, 0)),
            pl.BlockSpec((NC, _RB, 16), lambda i: (0, i, 0)),
        ],
        out_specs=pl.BlockSpec((_RB, D_OUT), lambda i: (i, 0)),
        out_shape=jax.ShapeDtypeStruct((N, D_OUT), jnp.float32),
    )(s_all, s_all, W1, deg2)


def _mmc(u_all, deg2):
    return pl.pallas_call(
        _mmc_body,
        grid=(N // _RB,),
        in_specs=[
            pl.BlockSpec((1, _RB, D_OUT), lambda i: (0, i, 0)),
            pl.BlockSpec((1, _RB, D_OUT), lambda i: (1, i, 0)),
            pl.BlockSpec((NC, _RB, 16), lambda i: (0, i, 0)),
        ],
        out_specs=pl.BlockSpec((_RB, D_OUT), lambda i: (i, 0)),
        out_shape=jax.ShapeDtypeStruct((N, D_OUT), jnp.float32),
    )(u_all, u_all, deg2)


def _mm3(z):
    ncb = pl.cdiv(N, _CB)
    return pl.pallas_call(
        _mm3_body,
        grid=(N // _RB, ncb),
        in_specs=[
            pl.BlockSpec((_RB, D_OUT), lambda i, j: (i, 0)),
            pl.BlockSpec((_CB, D_OUT), lambda i, j: (j, 0)),
        ],
        out_specs=pl.BlockSpec((_RB, _CB), lambda i, j: (i, j)),
        out_shape=jax.ShapeDtypeStruct((N, N), jnp.float32),
    )(z, z)


def kernel(h, edge_index, W0, W1):
    ei4 = edge_index.reshape(2, NS, NB, B)
    ones16 = jnp.ones((B, 16), jnp.float32)
    zz16 = jnp.zeros((RPT, 16), jnp.float32)
    zz64 = jnp.zeros((RPT, D_OUT), jnp.float32)
    zz128 = jnp.zeros((RPT, 128), jnp.float32)
    deg2 = _deg_kernel(ei4, ones16, zz16)
    x0, x1 = _mm1(h, W0, deg2)
    s_all = _spmm1(x0, x1, ei4, zz128)
    t2 = _mm2(s_all, W1, deg2)
    u_all = _spmm2(t2, ei4, zz64)
    z = _mmc(u_all, deg2)
    return _mm3(z)
